# baseline (speedup 1.0000x reference)
import jax
import jax.numpy as jnp
from jax import lax
from jax.experimental import pallas as pl
from jax.experimental.pallas import tpu as pltpu
from jax.experimental.pallas import tpu_sc as plsc

_NC = 2
_NS = 16
_NW = _NC * _NS
_D = 128
_L = 16
_W32 = _D // 2
_C = 80


def _tree_sum(terms):
    while len(terms) > 1:
        terms = [terms[i] + terms[i + 1] for i in range(0, len(terms) - 1, 2)] \
            + ([terms[-1]] if len(terms) % 2 else [])
    return terms[0]


def _sc_body(h_hbm, s_hbm, r_hbm, o_hbm, w_hbm, out_hbm,
             sidx, ridx, oidx, bh, bw, bo, outv, tmata, tmatb,
             gsema, gsemb):
    n = s_hbm.shape[0]
    t_per = n // _NW
    nch = t_per // _C
    wid = lax.axis_index("s") * _NC + lax.axis_index("c")
    gbase = wid * t_per

    pltpu.sync_copy(s_hbm.at[pl.ds(gbase, t_per)], sidx)
    pltpu.sync_copy(r_hbm.at[pl.ds(gbase, t_per)], ridx)
    pltpu.sync_copy(o_hbm.at[pl.ds(gbase, t_per)], oidx)

    gsems = (gsema, gsemb)
    tmats = (tmata, tmatb)
    lane = lax.broadcasted_iota(jnp.int32, (_L,), 0)

    def stage(g, sel):
        base = g * _C
        pltpu.async_copy(h_hbm.at[sidx.at[pl.ds(base, _C)]], bh.at[sel],
                         gsems[sel])
        pltpu.async_copy(w_hbm.at[ridx.at[pl.ds(base, _C)]], bw.at[sel],
                         gsems[sel])
        pltpu.async_copy(h_hbm.at[oidx.at[pl.ds(base, _C)]], bo.at[sel],
                         gsems[sel])

    def drain_gather(sel):
        pltpu.make_async_copy(h_hbm.at[sidx.at[pl.ds(0, _C)]], bh.at[sel],
                              gsems[sel]).wait()
        pltpu.make_async_copy(w_hbm.at[ridx.at[pl.ds(0, _C)]], bw.at[sel],
                              gsems[sel]).wait()
        pltpu.make_async_copy(h_hbm.at[oidx.at[pl.ds(0, _C)]], bo.at[sel],
                              gsems[sel]).wait()

    def group_scatter(hs, wr, ho, gi):
        tm = tmats[gi % 2]
        for j in range(_L):
            row = gi * _L + j
            terms = []
            for k in range(_D // (2 * _L)):
                a = plsc.bitcast(hs[row, pl.ds(k * _L, _L)], jnp.bfloat16)
                b_ = plsc.bitcast(wr[row, pl.ds(k * _L, _L)], jnp.bfloat16)
                c = plsc.bitcast(ho[row, pl.ds(k * _L, _L)], jnp.bfloat16)
                p0, p1 = plsc.unpack(a * b_ * c,
                                     format=plsc.PackFormat.INTERLEAVED)
                terms += [p0, p1]
            plsc.store_scatter(tm, [lane, jnp.full((_L,), j, jnp.int32)],
                               _tree_sum(terms))

    def group_sum(g, gi):
        tm = tmats[gi % 2]
        res = _tree_sum([tm[l, :] for l in range(_L)])
        outv[pl.ds(g * _C + gi * _L, _L)] = res

    stage(0, 0)

    @pl.loop(0, nch)
    def _chunk(g):
        par = g % 2
        more = g + 1 < nch

        @pl.when(jnp.logical_and(more, par == 0))
        def _():
            stage(g + 1, 1)

        @pl.when(jnp.logical_and(more, par == 1))
        def _():
            stage(g + 1, 0)

        @pl.when(par == 0)
        def _():
            drain_gather(0)

        @pl.when(par == 1)
        def _():
            drain_gather(1)

        hs = bh.at[par]
        wr = bw.at[par]
        ho = bo.at[par]
        ngr = _C // _L
        for gi in range(ngr):
            group_scatter(hs, wr, ho, gi)
            if gi > 0:
                group_sum(g, gi - 1)
        group_sum(g, ngr - 1)

    pltpu.sync_copy(outv, out_hbm.at[pl.ds(gbase, t_per)])


def kernel(h, triplets, W):
    n = triplets.shape[0]
    assert n % (_NW * _C) == 0
    s = triplets[:, 0]
    r = triplets[:, 1]
    o = triplets[:, 2]
    t_per = n // _NW
    mesh = plsc.VectorSubcoreMesh(
        core_axis_name="c", subcore_axis_name="s",
        num_cores=_NC, num_subcores=_NS)

    run = pl.kernel(
        _sc_body,
        out_type=jax.ShapeDtypeStruct((n,), jnp.float32),
        mesh=mesh,
        compiler_params=pltpu.CompilerParams(needs_layout_passes=False,
                                             use_tc_tiling_on_sc=False),
        scratch_types=[
            pltpu.VMEM((t_per,), jnp.int32),
            pltpu.VMEM((t_per,), jnp.int32),
            pltpu.VMEM((t_per,), jnp.int32),
            pltpu.VMEM((2, _C, _W32), jnp.int32),
            pltpu.VMEM((2, _C, _W32), jnp.int32),
            pltpu.VMEM((2, _C, _W32), jnp.int32),
            pltpu.VMEM((t_per,), jnp.float32),
            pltpu.VMEM((_L, _L), jnp.float32),
            pltpu.VMEM((_L, _L), jnp.float32),
            pltpu.SemaphoreType.DMA,
            pltpu.SemaphoreType.DMA,
        ],
    )

    def pack_bf16(x):
        xb = x.astype(jnp.bfloat16)
        return lax.bitcast_convert_type(
            xb.reshape(xb.shape[0], xb.shape[1] // 2, 2), jnp.int32)

    return run(pack_bf16(h), s, r, o, pack_bf16(W))
